# baseline (speedup 1.0000x reference)
import functools

import jax
import jax.numpy as jnp
from jax import lax
from jax.experimental import pallas as pl
from jax.experimental.pallas import tpu as pltpu
from jax.experimental.pallas import tpu_sc as plsc

D_MODEL = 1024
SEQ = 2048

_NC = 2
_NS = 16
_NW = _NC * _NS
_ROWS = SEQ // _NW

_mesh = plsc.VectorSubcoreMesh(core_axis_name="c", subcore_axis_name="s")

_NCHUNK = 8
_CROWS = _ROWS // _NCHUNK


@functools.partial(
    pl.kernel,
    mesh=_mesh,
    out_type=jax.ShapeDtypeStruct((SEQ, D_MODEL), jnp.float32),
    scratch_types=(
        [pltpu.VMEM((_CROWS, D_MODEL), jnp.float32) for _ in range(_NCHUNK)]
        + [pltpu.SemaphoreType.DMA for _ in range(2 * _NCHUNK)]
    ),
)
def _pos_copy(table_hbm, out_hbm, *scratch):
    bufs = scratch[:_NCHUNK]
    gsems = scratch[_NCHUNK : 2 * _NCHUNK]
    ssems = scratch[2 * _NCHUNK :]
    wid = lax.axis_index("s") * _NC + lax.axis_index("c")
    base = wid * _ROWS
    gets = [
        pltpu.async_copy(
            table_hbm.at[pl.ds(base + i * _CROWS, _CROWS)], bufs[i], gsems[i]
        )
        for i in range(_NCHUNK)
    ]
    puts = []
    for i in range(_NCHUNK):
        gets[i].wait()
        puts.append(
            pltpu.async_copy(
                bufs[i], out_hbm.at[pl.ds(base + i * _CROWS, _CROWS)], ssems[i]
            )
        )
    for p in puts:
        p.wait()


def kernel(x, pos_table):
    seq_len = x.shape[1]
    out = _pos_copy(pos_table)
    return out[None, :seq_len]


# speedup vs baseline: 1.0189x; 1.0189x over previous
import functools

import jax
import jax.numpy as jnp
from jax import lax
from jax.experimental import pallas as pl
from jax.experimental.pallas import tpu as pltpu
from jax.experimental.pallas import tpu_sc as plsc

D_MODEL = 1024
SEQ = 2048

_NC = 2
_NS = 16
_NW = _NC * _NS
_ROWS = SEQ // _NW

_mesh = plsc.VectorSubcoreMesh(core_axis_name="c", subcore_axis_name="s")

_NCHUNK = 2
_CROWS = _ROWS // _NCHUNK


@functools.partial(
    pl.kernel,
    mesh=_mesh,
    out_type=jax.ShapeDtypeStruct((SEQ, D_MODEL), jnp.float32),
    scratch_types=(
        [pltpu.VMEM((_CROWS, D_MODEL), jnp.float32) for _ in range(_NCHUNK)]
        + [pltpu.SemaphoreType.DMA for _ in range(2 * _NCHUNK)]
    ),
)
def _pos_copy(table_hbm, out_hbm, *scratch):
    bufs = scratch[:_NCHUNK]
    gsems = scratch[_NCHUNK : 2 * _NCHUNK]
    ssems = scratch[2 * _NCHUNK :]
    wid = lax.axis_index("s") * _NC + lax.axis_index("c")
    base = wid * _ROWS
    gets = [
        pltpu.async_copy(
            table_hbm.at[pl.ds(base + i * _CROWS, _CROWS)], bufs[i], gsems[i]
        )
        for i in range(_NCHUNK)
    ]
    puts = []
    for i in range(_NCHUNK):
        gets[i].wait()
        puts.append(
            pltpu.async_copy(
                bufs[i], out_hbm.at[pl.ds(base + i * _CROWS, _CROWS)], ssems[i]
            )
        )
    for p in puts:
        p.wait()


def kernel(x, pos_table):
    seq_len = x.shape[1]
    out = _pos_copy(pos_table)
    return out[None, :seq_len]
